# baseline (speedup 1.0000x reference)
import functools

import jax
import jax.numpy as jnp
from jax import lax
from jax.experimental import pallas as pl
from jax.experimental.pallas import tpu as pltpu
from jax.experimental.pallas import tpu_sc as plsc

N = 10000
E = 320000
R = 32
NB = 8
D = 128
E_PER_R = E // R
NBLK_A = 5
BA = N // NBLK_A
NBLK = 10
BN = N // NBLK
CK = 40
EH = E_PER_R // 2
NCH = EH // CK
SRCW = 5008
IBD = 25
NSLOT = 4
NS = 16
RPT = 624
ZROWS = 48


RH = R // 2


def _a_body(att_ref, basis_ref, x_ref, z_ref, w_scr, xb_scr):
    nb = pl.program_id(0)
    rp = pl.program_id(1)

    @pl.when((nb == 0) & (rp == 0))
    def _():
        for rr in range(RH):
            w = att_ref[rr, 0] * basis_ref[0]
            for b in range(1, NB):
                w = w + att_ref[rr, b] * basis_ref[b]
            half = (rr % 2) * D
            w_scr[rr // 2, :, pl.ds(half, D)] = w.astype(jnp.bfloat16)

    @pl.when(rp == 0)
    def _():
        xb_scr[...] = x_ref[...].astype(jnp.bfloat16)

    zz = jnp.dot(xb_scr[...], w_scr[rp], preferred_element_type=jnp.float32)
    z_ref[0] = zz[:, :D]
    z_ref[1] = zz[:, D:]


def _compute_z(att, basis, x):
    return pl.pallas_call(
        _a_body,
        grid=(NBLK_A, RH // 2),
        in_specs=[
            pl.BlockSpec(memory_space=pltpu.SMEM),
            pl.BlockSpec((NB, D, D), lambda nb, rp: (0, 0, 0)),
            pl.BlockSpec((BA, D), lambda nb, rp: (nb, 0)),
        ],
        out_specs=pl.BlockSpec((2, BA, D), lambda nb, rp: (rp, nb, 0)),
        out_shape=jax.ShapeDtypeStruct((RH, N, D), jnp.float32),
        scratch_shapes=[
            pltpu.VMEM((RH // 2, D, 2 * D), jnp.bfloat16),
            pltpu.VMEM((BA, D), jnp.bfloat16),
        ],
    )(att, basis, x)


def _p_body(ei_ref, src_ref, dst_ref):
    src_ref[...] = ei_ref[0]
    dst_ref[...] = ei_ref[1]


def _prep_idx(edge_index):
    return pl.pallas_call(
        _p_body,
        out_shape=[
            jax.ShapeDtypeStruct((E,), jnp.int32),
            jax.ShapeDtypeStruct((E,), jnp.int32),
        ],
    )(edge_index)


def _make_sc_body(r0, chained):
    def _sc_body(zflat, srcf, dst4, *rest):
        if chained:
            (pin, out, src_blk, dst_blk, rows_a, rows_b, rows_c, rows_d,
             acc_sh, sem_a, sem_b, sem_c, sem_d, sem_id) = rest
        else:
            pin = None
            (out, src_blk, dst_blk, rows_a, rows_b, rows_c, rows_d,
             acc_sh, sem_a, sem_b, sem_c, sem_d, sem_id) = rest
        cid = lax.axis_index("c")
        sid = lax.axis_index("s")
        rl = sid
        h = cid

        row0 = pl.multiple_of(sid * RPT, 8)
        if chained:
            pltpu.sync_copy(pin.at[cid, pl.ds(row0, RPT)],
                            acc_sh.at[pl.ds(row0, RPT)])

            @pl.when(sid == NS - 1)
            def _():
                tail = N - NS * RPT
                pltpu.sync_copy(pin.at[cid, pl.ds(NS * RPT, tail)],
                                acc_sh.at[pl.ds(NS * RPT, tail)])
        else:
            zero16 = jnp.zeros((16,), jnp.float32)

            def _zrow(i, carry):
                for j in range(D // 16):
                    rows_a[i, pl.ds(j * 16, 16)] = zero16
                return carry

            lax.fori_loop(0, CK, _zrow, 0)
            for k in range(RPT // ZROWS):
                pltpu.sync_copy(rows_a.at[pl.ds(0, ZROWS)],
                                acc_sh.at[pl.ds(row0 + k * ZROWS, ZROWS)])

            @pl.when(sid == NS - 1)
            def _():
                pltpu.sync_copy(rows_a.at[pl.ds(0, 16)],
                                acc_sh.at[pl.ds(NS * RPT, N - NS * RPT)])

        plsc.subcore_barrier()

        rbase = rl * N
        ebase = (r0 + rl) * E_PER_R + h * EH
        seg = (r0 + rl) * 2 + h

        pltpu.sync_copy(srcf.at[pl.ds(ebase, EH)], src_blk.at[pl.ds(0, EH)])

        def _radd(k, carry):
            sl = pl.ds(k * 16, 16)
            src_blk[sl] = src_blk[sl] + rbase
            return carry

        lax.fori_loop(0, SRCW // 16, _radd, 0)
        pltpu.sync_copy(dst4.at[seg, 0], dst_blk.at[0])
        pltpu.async_copy(dst4.at[seg, 1], dst_blk.at[1], sem_id)

        def _fire_g(slot, sem, c):
            off = pl.multiple_of(c * CK, 8)
            pltpu.async_copy(zflat.at[src_blk.at[pl.ds(off, CK)]], slot, sem)

        def _fire_s(slot, sem, c):
            pltpu.async_copy(slot,
                             acc_sh.at[dst_blk.at[(c // IBD) % 2, c % IBD]],
                             sem, add=True)

        def _drain_rows(sem, slot):
            pltpu.make_async_copy(zflat.at[pl.ds(0, CK)], slot, sem).wait()

        slots = (rows_a, rows_b, rows_c, rows_d)
        sems = (sem_a, sem_b, sem_c, sem_d)
        _fire_g(rows_a, sem_a, 0)
        _fire_g(rows_b, sem_b, 1)

        def _step(i, k):
            c = 4 * i + k
            s0 = k % NSLOT
            s2 = (k + 2) % NSLOT

            @pl.when(c >= 2)
            def _():
                _drain_rows(sems[s2], slots[s2])

            @pl.when((c % IBD == 1) & (c >= 26) & (c <= NCH - IBD * 2 + 1))
            def _():
                nb2 = c // IBD + 1
                pltpu.async_copy(dst4.at[seg, nb2], dst_blk.at[nb2 % 2],
                                 sem_id)

            @pl.when(((c + 2) % IBD == 0) & (c >= IBD - 2)
                     & (c <= NCH - IBD - 2))
            def _():
                pltpu.make_async_copy(dst4.at[seg, 0], dst_blk.at[0],
                                      sem_id).wait()

            @pl.when(c + 2 < NCH)
            def _():
                _fire_g(slots[s2], sems[s2], c + 2)

            _drain_rows(sems[s0], slots[s0])
            _fire_s(slots[s0], sems[s0], c)

        def _quad(i, carry):
            for k in range(NSLOT):
                _step(i, k)
            return carry

        lax.fori_loop(0, (NCH - 1) // NSLOT, _quad, 0)
        _drain_rows(sems[2], slots[2])
        _drain_rows(sems[0], slots[0])
        _fire_s(slots[0], sems[0], NCH - 1)
        _drain_rows(sems[3], slots[3])
        _drain_rows(sems[0], slots[0])
        plsc.subcore_barrier()

        pltpu.sync_copy(acc_sh.at[pl.ds(row0, RPT)],
                        out.at[cid, pl.ds(row0, RPT)])

        @pl.when(sid == NS - 1)
        def _():
            tail = N - NS * RPT
            pltpu.sync_copy(acc_sh.at[pl.ds(NS * RPT, tail)],
                            out.at[cid, pl.ds(NS * RPT, tail)])

    return _sc_body


def _sc_scatter(zflat, srcf, dst4, r0, pin=None):
    mesh = plsc.VectorSubcoreMesh(core_axis_name="c", subcore_axis_name="s")
    k = functools.partial(
        pl.kernel,
        out_type=jax.ShapeDtypeStruct((2, N, D), jnp.float32),
        mesh=mesh,
        scratch_types=[
            pltpu.VMEM((SRCW,), jnp.int32),
            pltpu.VMEM((2, IBD, CK), jnp.int32),
            pltpu.VMEM((CK, D), jnp.float32),
            pltpu.VMEM((CK, D), jnp.float32),
            pltpu.VMEM((CK, D), jnp.float32),
            pltpu.VMEM((CK, D), jnp.float32),
            pltpu.VMEM_SHARED((N, D), jnp.float32),
            pltpu.SemaphoreType.DMA,
            pltpu.SemaphoreType.DMA,
            pltpu.SemaphoreType.DMA,
            pltpu.SemaphoreType.DMA,
            pltpu.SemaphoreType.DMA,
        ],
    )(_make_sc_body(r0, pin is not None))
    if pin is None:
        return k(zflat, srcf, dst4)
    return k(zflat, srcf, dst4, pin)


def _b_body(p_ref, x_ref, dest_ref, root_ref, o_ref):
    s = p_ref[0] + p_ref[1]
    o_ref[...] = s / dest_ref[...] + jnp.dot(
        x_ref[...], root_ref[...], preferred_element_type=jnp.float32)


def _finish(parts, x, dest, root):
    return pl.pallas_call(
        _b_body,
        grid=(NBLK,),
        in_specs=[
            pl.BlockSpec((2, BN, D), lambda i: (0, i, 0)),
            pl.BlockSpec((BN, D), lambda i: (i, 0)),
            pl.BlockSpec((BN, 1), lambda i: (i, 0)),
            pl.BlockSpec((D, D), lambda i: (0, 0)),
        ],
        out_specs=pl.BlockSpec((BN, D), lambda i: (i, 0)),
        out_shape=jax.ShapeDtypeStruct((N, D), jnp.float32),
    )(parts, x, dest, root)


def kernel(x, edge_index, basis, att, root, dest_dict):
    srcf, dstf = _prep_idx(edge_index)
    dst4 = dstf.reshape(2 * R, NCH // IBD, IBD, CK)
    z1 = _compute_z(att[:RH], basis, x)
    p1 = _sc_scatter(z1.reshape(RH * N, D), srcf, dst4, 0)
    z2 = _compute_z(att[RH:], basis, x)
    p2 = _sc_scatter(z2.reshape(RH * N, D), srcf, dst4, RH, pin=p1)
    return _finish(p2, x, dest_dict, root)


# speedup vs baseline: 1.0060x; 1.0060x over previous
import functools

import jax
import jax.numpy as jnp
from jax import lax
from jax.experimental import pallas as pl
from jax.experimental.pallas import tpu as pltpu
from jax.experimental.pallas import tpu_sc as plsc

N = 10000
E = 320000
R = 32
NB = 8
D = 128
E_PER_R = E // R
NBLK_A = 5
BA = N // NBLK_A
NBLK = 10
BN = N // NBLK
CK = 40
EH = E_PER_R // 2
NCH = EH // CK
SRCW = 5008
IBD = 25
NSLOT = 4
NS = 16
RPT = 624
ZROWS = 48


RH = R // 2


def _a_body(att_ref, basis_ref, x_ref, z_ref, w_scr, xb_scr):
    nb = pl.program_id(0)
    rp = pl.program_id(1)

    @pl.when((nb == 0) & (rp == 0))
    def _():
        for rr in range(RH):
            w = att_ref[rr, 0] * basis_ref[0]
            for b in range(1, NB):
                w = w + att_ref[rr, b] * basis_ref[b]
            half = (rr % 2) * D
            w_scr[rr // 2, :, pl.ds(half, D)] = w.astype(jnp.bfloat16)

    @pl.when(rp == 0)
    def _():
        xb_scr[...] = x_ref[...].astype(jnp.bfloat16)

    zz = jnp.dot(xb_scr[...], w_scr[rp], preferred_element_type=jnp.float32)
    z_ref[0] = zz[:, :D]
    z_ref[1] = zz[:, D:]


def _compute_z(att, basis, x):
    return pl.pallas_call(
        _a_body,
        grid=(NBLK_A, RH // 2),
        in_specs=[
            pl.BlockSpec(memory_space=pltpu.SMEM),
            pl.BlockSpec((NB, D, D), lambda nb, rp: (0, 0, 0)),
            pl.BlockSpec((BA, D), lambda nb, rp: (nb, 0)),
        ],
        out_specs=pl.BlockSpec((2, BA, D), lambda nb, rp: (rp, nb, 0)),
        out_shape=jax.ShapeDtypeStruct((RH, N, D), jnp.float32),
        scratch_shapes=[
            pltpu.VMEM((RH // 2, D, 2 * D), jnp.bfloat16),
            pltpu.VMEM((BA, D), jnp.bfloat16),
        ],
    )(att, basis, x)


def _p_body(ei_ref, src_ref, dst_ref):
    src_ref[...] = ei_ref[0]
    dst_ref[...] = ei_ref[1]


def _prep_idx(edge_index):
    return pl.pallas_call(
        _p_body,
        out_shape=[
            jax.ShapeDtypeStruct((E,), jnp.int32),
            jax.ShapeDtypeStruct((E,), jnp.int32),
        ],
    )(edge_index)


def _make_sc_body(r0):
    def _sc_body(zflat, srcf, dst4, out, src_blk, dst_blk,
                 rows_a, rows_b, rows_c, rows_d, acc_sh,
                 sem_a, sem_b, sem_c, sem_d, sem_id):
        cid = lax.axis_index("c")
        sid = lax.axis_index("s")
        rl = sid
        h = cid

        zero16 = jnp.zeros((16,), jnp.float32)

        def _zrow(i, carry):
            for j in range(D // 16):
                rows_a[i, pl.ds(j * 16, 16)] = zero16
            return carry

        lax.fori_loop(0, CK, _zrow, 0)
        row0 = pl.multiple_of(sid * RPT, 8)
        for k in range(RPT // ZROWS):
            pltpu.sync_copy(rows_a.at[pl.ds(0, ZROWS)],
                            acc_sh.at[pl.ds(row0 + k * ZROWS, ZROWS)])

        @pl.when(sid == NS - 1)
        def _():
            pltpu.sync_copy(rows_a.at[pl.ds(0, 16)],
                            acc_sh.at[pl.ds(NS * RPT, N - NS * RPT)])

        plsc.subcore_barrier()

        rbase = rl * N
        ebase = (r0 + rl) * E_PER_R + h * EH
        seg = (r0 + rl) * 2 + h

        pltpu.sync_copy(srcf.at[pl.ds(ebase, EH)], src_blk.at[pl.ds(0, EH)])

        def _radd(k, carry):
            sl = pl.ds(k * 16, 16)
            src_blk[sl] = src_blk[sl] + rbase
            return carry

        lax.fori_loop(0, SRCW // 16, _radd, 0)
        pltpu.sync_copy(dst4.at[seg, 0], dst_blk.at[0])
        pltpu.async_copy(dst4.at[seg, 1], dst_blk.at[1], sem_id)

        def _fire_g(slot, sem, c):
            off = pl.multiple_of(c * CK, 8)
            pltpu.async_copy(zflat.at[src_blk.at[pl.ds(off, CK)]], slot, sem)

        def _fire_s(slot, sem, c):
            pltpu.async_copy(slot,
                             acc_sh.at[dst_blk.at[(c // IBD) % 2, c % IBD]],
                             sem, add=True)

        def _drain_rows(sem, slot):
            pltpu.make_async_copy(zflat.at[pl.ds(0, CK)], slot, sem).wait()

        slots = (rows_a, rows_b, rows_c, rows_d)
        sems = (sem_a, sem_b, sem_c, sem_d)
        _fire_g(rows_a, sem_a, 0)
        _fire_g(rows_b, sem_b, 1)

        def _step(i, k):
            c = 4 * i + k
            s0 = k % NSLOT
            s2 = (k + 2) % NSLOT

            @pl.when(c >= 2)
            def _():
                _drain_rows(sems[s2], slots[s2])

            @pl.when((c % IBD == 1) & (c >= 26) & (c <= NCH - IBD * 2 + 1))
            def _():
                nb2 = c // IBD + 1
                pltpu.async_copy(dst4.at[seg, nb2], dst_blk.at[nb2 % 2],
                                 sem_id)

            @pl.when(((c + 2) % IBD == 0) & (c >= IBD - 2)
                     & (c <= NCH - IBD - 2))
            def _():
                pltpu.make_async_copy(dst4.at[seg, 0], dst_blk.at[0],
                                      sem_id).wait()

            @pl.when(c + 2 < NCH)
            def _():
                _fire_g(slots[s2], sems[s2], c + 2)

            _drain_rows(sems[s0], slots[s0])
            _fire_s(slots[s0], sems[s0], c)

        def _quad(i, carry):
            for k in range(NSLOT):
                _step(i, k)
            return carry

        lax.fori_loop(0, (NCH - 1) // NSLOT, _quad, 0)
        _drain_rows(sems[2], slots[2])
        _drain_rows(sems[0], slots[0])
        _fire_s(slots[0], sems[0], NCH - 1)
        _drain_rows(sems[3], slots[3])
        _drain_rows(sems[0], slots[0])
        plsc.subcore_barrier()

        pltpu.sync_copy(acc_sh.at[pl.ds(row0, RPT)],
                        out.at[cid, pl.ds(row0, RPT)])

        @pl.when(sid == NS - 1)
        def _():
            tail = N - NS * RPT
            pltpu.sync_copy(acc_sh.at[pl.ds(NS * RPT, tail)],
                            out.at[cid, pl.ds(NS * RPT, tail)])

    return _sc_body


def _sc_scatter(zflat, srcf, dst4, r0):
    mesh = plsc.VectorSubcoreMesh(core_axis_name="c", subcore_axis_name="s")
    k = functools.partial(
        pl.kernel,
        out_type=jax.ShapeDtypeStruct((2, N, D), jnp.float32),
        mesh=mesh,
        scratch_types=[
            pltpu.VMEM((SRCW,), jnp.int32),
            pltpu.VMEM((2, IBD, CK), jnp.int32),
            pltpu.VMEM((CK, D), jnp.float32),
            pltpu.VMEM((CK, D), jnp.float32),
            pltpu.VMEM((CK, D), jnp.float32),
            pltpu.VMEM((CK, D), jnp.float32),
            pltpu.VMEM_SHARED((N, D), jnp.float32),
            pltpu.SemaphoreType.DMA,
            pltpu.SemaphoreType.DMA,
            pltpu.SemaphoreType.DMA,
            pltpu.SemaphoreType.DMA,
            pltpu.SemaphoreType.DMA,
        ],
    )(_make_sc_body(r0))
    return k(zflat, srcf, dst4)


def _b_body(p1_ref, p2_ref, x_ref, dest_ref, root_ref, o_ref):
    s = (p1_ref[0] + p1_ref[1]) + (p2_ref[0] + p2_ref[1])
    o_ref[...] = s / dest_ref[...] + jnp.dot(
        x_ref[...], root_ref[...], preferred_element_type=jnp.float32)


def _finish(p1, p2, x, dest, root):
    return pl.pallas_call(
        _b_body,
        grid=(NBLK,),
        in_specs=[
            pl.BlockSpec((2, BN, D), lambda i: (0, i, 0)),
            pl.BlockSpec((2, BN, D), lambda i: (0, i, 0)),
            pl.BlockSpec((BN, D), lambda i: (i, 0)),
            pl.BlockSpec((BN, 1), lambda i: (i, 0)),
            pl.BlockSpec((D, D), lambda i: (0, 0)),
        ],
        out_specs=pl.BlockSpec((BN, D), lambda i: (i, 0)),
        out_shape=jax.ShapeDtypeStruct((N, D), jnp.float32),
    )(p1, p2, x, dest, root)


def kernel(x, edge_index, basis, att, root, dest_dict):
    srcf, dstf = _prep_idx(edge_index)
    dst4 = dstf.reshape(2 * R, NCH // IBD, IBD, CK)
    z1 = _compute_z(att[:RH], basis, x)
    p1 = _sc_scatter(z1.reshape(RH * N, D), srcf, dst4, 0)
    z2 = _compute_z(att[RH:], basis, x)
    p2 = _sc_scatter(z2.reshape(RH * N, D), srcf, dst4, RH)
    return _finish(p1, p2, x, dest_dict, root)
